# speedup vs baseline: 1.3445x; 1.3445x over previous
import jax
import jax.numpy as jnp
from jax import lax
from jax.experimental import pallas as pl
from jax.experimental.pallas import tpu as pltpu
from jax.experimental.pallas import tpu_sc as plsc

BATCH = 4
SEQ = 2048
D = 768
LANES = 16
NUM_WORKERS = 32
P = SEQ // NUM_WORKERS
CH = 32
NCH = P // CH
VREGS_PER_ROW = D // LANES


def _body(ids_hbm, wte_hbm, wpe_hbm, out_hbm, idx_v, wpe_v, buf, sem):
    wid = lax.axis_index("s") * 2 + lax.axis_index("c")
    pos0 = wid * P

    pltpu.sync_copy(wpe_hbm.at[pl.ds(pos0, P)], wpe_v)

    for b in range(BATCH):
        pltpu.sync_copy(ids_hbm.at[b, pl.ds(pos0, P)], idx_v.at[b])

    for b in range(BATCH):
        for k in range(NCH):
            pltpu.async_copy(
                wte_hbm.at[idx_v.at[b, pl.ds(k * CH, CH)]], buf, sem
            ).wait()

            def add_row(r, carry, _k=k):
                for j in range(VREGS_PER_ROW):
                    sl = pl.ds(j * LANES, LANES)
                    plsc.addupdate(buf.at[r, sl], wpe_v[_k * CH + r, sl])
                return carry

            lax.fori_loop(0, CH, add_row, 0)

            pltpu.sync_copy(buf, out_hbm.at[b, pl.ds(pos0 + k * CH, CH)])


@jax.jit
def _embed(input_ids, wte, wpe):
    mesh = plsc.VectorSubcoreMesh(core_axis_name="c", subcore_axis_name="s")
    return pl.kernel(
        _body,
        out_type=jax.ShapeDtypeStruct((BATCH, SEQ, D), jnp.float32),
        mesh=mesh,
        scratch_types=[
            pltpu.VMEM((BATCH, P), jnp.int32),
            pltpu.VMEM((P, D), jnp.float32),
            pltpu.VMEM((CH, D), jnp.float32),
            pltpu.SemaphoreType.DMA,
        ],
    )(input_ids, wte, wpe)


def kernel(input_ids, wte, wpe):
    return _embed(input_ids, wte, wpe)


# speedup vs baseline: 1.4868x; 1.1058x over previous
import jax
import jax.numpy as jnp
from jax import lax
from jax.experimental import pallas as pl
from jax.experimental.pallas import tpu as pltpu
from jax.experimental.pallas import tpu_sc as plsc

BATCH = 4
SEQ = 2048
D = 768
LANES = 16
NUM_WORKERS = 32
P = SEQ // NUM_WORKERS
CH = 32
NCH = P // CH
VREGS_PER_ROW = D // LANES


NB = 3
NCHUNK = BATCH * NCH


def _body(ids_hbm, wte_hbm, wpe_hbm, out_hbm, idx_v, wpe_v,
          buf0, buf1, buf2, g0, g1, g2, s0, s1, s2):
    bufs = (buf0, buf1, buf2)
    gsem = (g0, g1, g2)
    ssem = (s0, s1, s2)

    wid = lax.axis_index("s") * 2 + lax.axis_index("c")
    pos0 = wid * P

    pltpu.sync_copy(wpe_hbm.at[pl.ds(pos0, P)], wpe_v)

    for b in range(BATCH):
        pltpu.sync_copy(ids_hbm.at[b, pl.ds(pos0, P)], idx_v.at[b])

    def gather(i):
        b, k = divmod(i, NCH)
        return pltpu.async_copy(
            wte_hbm.at[idx_v.at[b, pl.ds(k * CH, CH)]],
            bufs[i % NB], gsem[i % NB])

    def store(i):
        b, k = divmod(i, NCH)
        return pltpu.async_copy(
            bufs[i % NB], out_hbm.at[b, pl.ds(pos0 + k * CH, CH)],
            ssem[i % NB])

    def add_chunk(i):
        buf = bufs[i % NB]
        k = i % NCH

        def add_row(r, carry):
            for j in range(VREGS_PER_ROW):
                sl = pl.ds(j * LANES, LANES)
                plsc.addupdate(buf.at[r, sl], wpe_v[k * CH + r, sl])
            return carry

        lax.fori_loop(0, CH, add_row, 0)

    g = {0: gather(0), 1: gather(1)}
    s = {}
    for i in range(NCHUNK):
        g[i].wait()
        add_chunk(i)
        s[i] = store(i)
        nxt = i + 2
        if nxt < NCHUNK:
            if nxt - NB in s:
                s.pop(nxt - NB).wait()
            g[nxt] = gather(nxt)
    for i in sorted(s):
        s[i].wait()


@jax.jit
def _embed(input_ids, wte, wpe):
    mesh = plsc.VectorSubcoreMesh(core_axis_name="c", subcore_axis_name="s")
    return pl.kernel(
        _body,
        out_type=jax.ShapeDtypeStruct((BATCH, SEQ, D), jnp.float32),
        mesh=mesh,
        scratch_types=[
            pltpu.VMEM((BATCH, P), jnp.int32),
            pltpu.VMEM((P, D), jnp.float32),
            pltpu.VMEM((CH, D), jnp.float32),
            pltpu.VMEM((CH, D), jnp.float32),
            pltpu.VMEM((CH, D), jnp.float32),
            pltpu.SemaphoreType.DMA,
            pltpu.SemaphoreType.DMA,
            pltpu.SemaphoreType.DMA,
            pltpu.SemaphoreType.DMA,
            pltpu.SemaphoreType.DMA,
            pltpu.SemaphoreType.DMA,
        ],
    )(input_ids, wte, wpe)


def kernel(input_ids, wte, wpe):
    return _embed(input_ids, wte, wpe)


# speedup vs baseline: 1.7072x; 1.1482x over previous
import jax
import jax.numpy as jnp
from jax import lax
from jax.experimental import pallas as pl
from jax.experimental.pallas import tpu as pltpu
from jax.experimental.pallas import tpu_sc as plsc

BATCH = 4
SEQ = 2048
D = 768
LANES = 16
NUM_WORKERS = 32
P = SEQ // NUM_WORKERS
CP = 8
NCHUNK = P // CP
NB = 3
VREGS_PER_ROW = D // LANES


def _body(ids_hbm, wte_hbm, wpe_hbm, out_hbm,
          idx_v, gbufs, wbufs, g0, g1, g2, w0, w1, w2, s0, s1, s2):
    gsem = (g0, g1, g2)
    wsem = (w0, w1, w2)
    ssem = (s0, s1, s2)

    wid = lax.axis_index("s") * 2 + lax.axis_index("c")
    pos0 = wid * P

    for b in range(BATCH):
        pltpu.sync_copy(ids_hbm.at[b, pl.ds(pos0, P)], idx_v.at[b])

    def loads(k):
        st = k % NB
        h = [pltpu.async_copy(
            wpe_hbm.at[pl.ds(pos0 + k * CP, CP)], wbufs.at[st], wsem[st])]
        for b in range(BATCH):
            h.append(pltpu.async_copy(
                wte_hbm.at[idx_v.at[b, pl.ds(k * CP, CP)]],
                gbufs.at[st, b], gsem[st]))
        return h

    def stores(k):
        st = k % NB
        return [pltpu.async_copy(
            gbufs.at[st, b], out_hbm.at[b, pl.ds(pos0 + k * CP, CP)],
            ssem[st]) for b in range(BATCH)]

    def add_chunk(k):
        st = k % NB
        def add_row(r, carry):
            for j in range(VREGS_PER_ROW):
                sl = pl.ds(j * LANES, LANES)
                v = wbufs[st, r, sl]
                for b in range(BATCH):
                    plsc.addupdate(gbufs.at[st, b, r, sl], v)
            return carry

        lax.fori_loop(0, CP, add_row, 0)

    live_loads = {k: loads(k) for k in range(min(2, NCHUNK))}
    live_stores = {}
    for k in range(NCHUNK):
        for h in live_loads.pop(k):
            h.wait()
        add_chunk(k)
        live_stores[k] = stores(k)
        nk = k + 2
        if nk < NCHUNK:
            for h in live_stores.pop(nk - NB, []):
                h.wait()
            live_loads[nk] = loads(nk)
    for k in sorted(live_stores):
        for h in live_stores[k]:
            h.wait()


@jax.jit
def _embed(input_ids, wte, wpe):
    mesh = plsc.VectorSubcoreMesh(core_axis_name="c", subcore_axis_name="s")
    return pl.kernel(
        _body,
        out_type=jax.ShapeDtypeStruct((BATCH, SEQ, D), jnp.float32),
        mesh=mesh,
        scratch_types=[
            pltpu.VMEM((BATCH, P), jnp.int32),
            pltpu.VMEM((NB, BATCH, CP, D), jnp.float32),
            pltpu.VMEM((NB, CP, D), jnp.float32),
            pltpu.SemaphoreType.DMA,
            pltpu.SemaphoreType.DMA,
            pltpu.SemaphoreType.DMA,
            pltpu.SemaphoreType.DMA,
            pltpu.SemaphoreType.DMA,
            pltpu.SemaphoreType.DMA,
            pltpu.SemaphoreType.DMA,
            pltpu.SemaphoreType.DMA,
            pltpu.SemaphoreType.DMA,
        ],
    )(input_ids, wte, wpe)


def kernel(input_ids, wte, wpe):
    return _embed(input_ids, wte, wpe)


# speedup vs baseline: 1.7507x; 1.0255x over previous
import jax
import jax.numpy as jnp
from jax import lax
from jax.experimental import pallas as pl
from jax.experimental.pallas import tpu as pltpu
from jax.experimental.pallas import tpu_sc as plsc

BATCH = 4
SEQ = 2048
D = 768
LANES = 16
NUM_WORKERS = 32
P = SEQ // NUM_WORKERS
CP = 8
NCHUNK = P // CP
NB = 3
VREGS_PER_ROW = D // LANES


def _body(ids_hbm, wte_hbm, wpe_hbm, out_hbm,
          idx_v, gbufs, wbufs, g0, g1, g2, w0, w1, w2, s0, s1, s2):
    gsem = (g0, g1, g2)
    wsem = (w0, w1, w2)
    ssem = (s0, s1, s2)

    wid = lax.axis_index("s") * 2 + lax.axis_index("c")
    pos0 = wid * P

    idx_copies = [
        pltpu.async_copy(ids_hbm.at[b, pl.ds(pos0, P)], idx_v.at[b], g0)
        for b in range(BATCH)
    ]
    for h in idx_copies:
        h.wait()

    def loads(k):
        st = k % NB
        h = [pltpu.async_copy(
            wpe_hbm.at[pl.ds(pos0 + k * CP, CP)], wbufs.at[st], wsem[st])]
        for b in range(BATCH):
            h.append(pltpu.async_copy(
                wte_hbm.at[idx_v.at[b, pl.ds(k * CP, CP)]],
                gbufs.at[st, b], gsem[st]))
        return h

    def stores(k):
        st = k % NB
        return [pltpu.async_copy(
            gbufs.at[st, b], out_hbm.at[b, pl.ds(pos0 + k * CP, CP)],
            ssem[st]) for b in range(BATCH)]

    def add_chunk(k):
        st = k % NB
        def add_row(r, carry):
            for j in range(VREGS_PER_ROW):
                sl = pl.ds(j * LANES, LANES)
                v = wbufs[st, r, sl]
                for b in range(BATCH):
                    plsc.addupdate(gbufs.at[st, b, r, sl], v)
            return carry

        lax.fori_loop(0, CP, add_row, 0)

    live_loads = {k: loads(k) for k in range(min(2, NCHUNK))}
    live_stores = {}
    for k in range(NCHUNK):
        for h in live_loads.pop(k):
            h.wait()
        add_chunk(k)
        live_stores[k] = stores(k)
        nk = k + 2
        if nk < NCHUNK:
            for h in live_stores.pop(nk - NB, []):
                h.wait()
            live_loads[nk] = loads(nk)
    for k in sorted(live_stores):
        for h in live_stores[k]:
            h.wait()


@jax.jit
def _embed(input_ids, wte, wpe):
    mesh = plsc.VectorSubcoreMesh(core_axis_name="c", subcore_axis_name="s")
    return pl.kernel(
        _body,
        out_type=jax.ShapeDtypeStruct((BATCH, SEQ, D), jnp.float32),
        mesh=mesh,
        scratch_types=[
            pltpu.VMEM((BATCH, P), jnp.int32),
            pltpu.VMEM((NB, BATCH, CP, D), jnp.float32),
            pltpu.VMEM((NB, CP, D), jnp.float32),
            pltpu.SemaphoreType.DMA,
            pltpu.SemaphoreType.DMA,
            pltpu.SemaphoreType.DMA,
            pltpu.SemaphoreType.DMA,
            pltpu.SemaphoreType.DMA,
            pltpu.SemaphoreType.DMA,
            pltpu.SemaphoreType.DMA,
            pltpu.SemaphoreType.DMA,
            pltpu.SemaphoreType.DMA,
        ],
    )(input_ids, wte, wpe)


def kernel(input_ids, wte, wpe):
    return _embed(input_ids, wte, wpe)
